# Initial kernel scaffold; baseline (speedup 1.0000x reference)
import jax
import jax.numpy as jnp
from jax.experimental import pallas as pl


def kernel(emb_s, emb_t, at_s, at_t, x_s, x_t, edge_index, edge_weight, W0, b0, W1, b1, W2, b2, ln0_g, ln0_b, ln1_g, ln1_b, fW0, fb0, fln_g, fln_b, fW1, fb1):
    raise NotImplementedError("write your pallas kernel here")


# speedup vs baseline: 7.3535x; 7.3535x over previous
import dataclasses
import functools

import jax
import jax.numpy as jnp
from jax import lax
from jax.experimental import pallas as pl
from jax.experimental.pallas import tpu as pltpu
from jax.experimental.pallas import tpu_sc as plsc

N_NODES = 10000
E_TOTAL = 320000
D = 128
W_G = 80
ROWS = E_TOTAL // W_G
N_TILES = 32
ROWS_PER_TILE = ROWS // N_TILES
ROWS_PER_SUB = ROWS // 16
BE = 2560
C_COEF = 160
BR = 16

_mesh = plsc.VectorSubcoreMesh(core_axis_name="c", subcore_axis_name="s")

_sc_cp = pltpu.CompilerParams()
if "needs_layout_passes" in pltpu.CompilerParams.__dataclass_fields__:
    _sc_cp = dataclasses.replace(_sc_cp, needs_layout_passes=False)


def _leaky(x):
    return jnp.where(x >= 0, x, 0.01 * x)


def _ln(x, g, b):
    m = jnp.mean(x, axis=-1, keepdims=True)
    v = jnp.mean((x - m) ** 2, axis=-1, keepdims=True)
    return (x - m) * lax.rsqrt(v + 1e-5) * g + b


def _pre_body(emb_ref, w_ref, tab_ref, ns_ref):
    e = emb_ref[...]
    tab_ref[...] = jnp.dot(e, w_ref[...], precision=lax.Precision.HIGHEST,
                           preferred_element_type=jnp.float32)
    ns_ref[...] = jnp.sum(e * e, axis=1)


def _pre(emb, w0part):
    n = emb.shape[0]
    return pl.pallas_call(
        _pre_body,
        out_shape=(jax.ShapeDtypeStruct((n, D), jnp.float32),
                   jax.ShapeDtypeStruct((n,), jnp.float32)),
    )(emb, w0part)


@functools.partial(
    pl.kernel, mesh=_mesh,
    out_type=(jax.ShapeDtypeStruct((E_TOTAL, D), jnp.float32),
              jax.ShapeDtypeStruct((E_TOTAL, D), jnp.float32)),
    scratch_types=[
        pltpu.VMEM((ROWS_PER_TILE, W_G), jnp.int32),
        pltpu.VMEM((ROWS_PER_TILE, W_G), jnp.int32),
        pltpu.VMEM((W_G, D), jnp.float32),
        pltpu.VMEM((W_G, D), jnp.float32),
        pltpu.SemaphoreType.DMA,
    ],
    compiler_params=_sc_cp,
)
def _gather(tabs_hbm, tabt_hbm, src_hbm, dst_hbm,
            gs_hbm, gt_hbm,
            idxs_v, idxt_v, bufs_v, buft_v, sem):
    wid = lax.axis_index("s") * 2 + lax.axis_index("c")
    base_r = wid * ROWS_PER_TILE
    pltpu.async_copy(src_hbm.at[wid], idxs_v, sem).wait()
    pltpu.async_copy(dst_hbm.at[wid], idxt_v, sem).wait()

    @pl.loop(0, ROWS_PER_TILE)
    def _row(j):
        e0 = (base_r + j) * W_G
        pltpu.async_copy(tabs_hbm.at[idxs_v.at[j]], bufs_v, sem).wait()
        pltpu.async_copy(bufs_v, gs_hbm.at[pl.ds(e0, W_G)], sem).wait()
        pltpu.async_copy(tabt_hbm.at[idxt_v.at[j]], buft_v, sem).wait()
        pltpu.async_copy(buft_v, gt_hbm.at[pl.ds(e0, W_G)], sem).wait()


@functools.partial(
    pl.kernel, mesh=_mesh,
    out_type=(jax.ShapeDtypeStruct((N_TILES, ROWS_PER_TILE, W_G), jnp.float32),
              jax.ShapeDtypeStruct((N_TILES, ROWS_PER_TILE, W_G), jnp.float32),
              jax.ShapeDtypeStruct((N_TILES, ROWS_PER_TILE, W_G), jnp.float32)),
    scratch_types=[
        pltpu.VMEM((ROWS_PER_TILE, W_G), jnp.int32),
        pltpu.VMEM((ROWS_PER_TILE, W_G), jnp.int32),
        pltpu.VMEM((N_NODES,), jnp.float32),
        pltpu.VMEM((N_NODES,), jnp.float32),
        pltpu.VMEM((N_NODES,), jnp.float32),
        pltpu.VMEM((N_NODES,), jnp.float32),
        pltpu.VMEM((ROWS_PER_TILE, W_G), jnp.float32),
        pltpu.VMEM((ROWS_PER_TILE, W_G), jnp.float32),
        pltpu.VMEM((ROWS_PER_TILE, W_G), jnp.float32),
        pltpu.SemaphoreType.DMA,
    ],
    compiler_params=_sc_cp,
)
def _scal(nss_hbm, nst_hbm, x1s_hbm, x1t_hbm, src_hbm, dst_hbm,
          nsum_hbm, xs1_hbm, xt1_hbm,
          idxs_v, idxt_v, nss_v, nst_v, x1s_v, x1t_v,
          nsum_v, xs1_v, xt1_v, sem):
    wid = lax.axis_index("s") * 2 + lax.axis_index("c")
    pltpu.async_copy(src_hbm.at[wid], idxs_v, sem).wait()
    pltpu.async_copy(dst_hbm.at[wid], idxt_v, sem).wait()
    pltpu.async_copy(nss_hbm, nss_v, sem).wait()
    pltpu.async_copy(nst_hbm, nst_v, sem).wait()
    pltpu.async_copy(x1s_hbm, x1s_v, sem).wait()
    pltpu.async_copy(x1t_hbm, x1t_v, sem).wait()

    @pl.loop(0, ROWS_PER_TILE)
    def _row(j):
        for k in range(W_G // 16):
            sl = pl.ds(k * 16, 16)
            i16s = idxs_v[j, sl]
            i16t = idxt_v[j, sl]
            nsum_v[j, sl] = (plsc.load_gather(nss_v, [i16s])
                             + plsc.load_gather(nst_v, [i16t]))
            xs1_v[j, sl] = plsc.load_gather(x1s_v, [i16s])
            xt1_v[j, sl] = plsc.load_gather(x1t_v, [i16t])

    pltpu.async_copy(nsum_v, nsum_hbm.at[wid], sem).wait()
    pltpu.async_copy(xs1_v, xs1_hbm.at[wid], sem).wait()
    pltpu.async_copy(xt1_v, xt1_hbm.at[wid], sem).wait()


def _fnn_body(gs_ref, gt_ref, n2s_ref, xs1_ref, xt1_ref, ats_ref, att_ref,
              ew_ref,
              w0x_ref, b0_ref, g0_ref, bb0_ref,
              w1t_ref, b1_ref, g1_ref, bb1_ref,
              w2_ref, b2_ref,
              y_ref):
    ps = gs_ref[...] + gt_ref[...]
    nsum = n2s_ref[...]
    xs1 = xs1_ref[...]
    xt1 = xt1_ref[...]
    asm = jnp.mean(ats_ref[...], axis=1, keepdims=True)
    atm = jnp.mean(att_ref[...], axis=1, keepdims=True)
    ew = ew_ref[...]
    w0x = w0x_ref[...]
    z = (ps + xs1 * w0x[0:1, :] + xt1 * w0x[1:2, :] + asm * w0x[2:3, :]
         + atm * w0x[3:4, :] + ew * w0x[4:5, :])
    n2 = nsum + xs1 * xs1 + xt1 * xt1 + asm * asm + atm * atm + ew * ew
    inv = 1.0 / jnp.maximum(jnp.sqrt(n2), 1e-12)
    z = z * inv + b0_ref[...]
    y1 = _ln(_leaky(z), g0_ref[...], bb0_ref[...])
    z2 = jnp.dot(y1, w1t_ref[...], precision=lax.Precision.HIGHEST,
                 preferred_element_type=jnp.float32)
    y2 = _ln(_leaky(z2 + b1_ref[...]), g1_ref[...], bb1_ref[...])
    yv = jnp.sum(y2 * w2_ref[...], axis=1, keepdims=True) + b2_ref[...]
    y_ref[...] = jnp.maximum(yv, 0.0)


def _fnn(gs, gt, n2s, xs1e, xt1e, ats1, att1, ew,
         w0x, b0, g0, bb0, w1t, b1, g1, bb1, w2, b2):
    nb = E_TOTAL // BE
    col = lambda: pl.BlockSpec((BE, 1), lambda i: (i, 0))
    wspec = lambda r, c: pl.BlockSpec((r, c), lambda i: (0, 0))
    return pl.pallas_call(
        _fnn_body,
        grid=(nb,),
        in_specs=[
            pl.BlockSpec((BE, D), lambda i: (i, 0)),
            pl.BlockSpec((BE, D), lambda i: (i, 0)),
            col(), col(), col(),
            pl.BlockSpec((BE, 8), lambda i: (i, 0)),
            pl.BlockSpec((BE, 8), lambda i: (i, 0)),
            col(),
            wspec(5, D), wspec(1, D), wspec(1, D), wspec(1, D),
            wspec(D, D), wspec(1, D), wspec(1, D), wspec(1, D),
            wspec(1, D), wspec(1, 1),
        ],
        out_specs=col(),
        out_shape=jax.ShapeDtypeStruct((E_TOTAL, 1), jnp.float32),
        compiler_params=pltpu.CompilerParams(
            dimension_semantics=("parallel",)),
    )(gs, gt, n2s, xs1e, xt1e, ats1, att1, ew,
      w0x, b0, g0, bb0, w1t, b1, g1, bb1, w2, b2)


@functools.partial(
    pl.kernel, mesh=_mesh,
    out_type=(jax.ShapeDtypeStruct((16, ROWS_PER_SUB, W_G), jnp.float32),
              jax.ShapeDtypeStruct((16, ROWS_PER_SUB, W_G), jnp.float32)),
    scratch_types=[
        pltpu.VMEM((ROWS_PER_SUB, W_G), jnp.float32),
        pltpu.VMEM((ROWS_PER_SUB, W_G), jnp.int32),
        pltpu.VMEM_SHARED((N_NODES,), jnp.float32),
        pltpu.VMEM((N_NODES,), jnp.float32),
        pltpu.VMEM((ROWS_PER_SUB, W_G), jnp.float32),
        pltpu.SemaphoreType.DMA,
    ],
    compiler_params=_sc_cp,
)
def _seg(y_hbm, src_hbm, dst_hbm, gi_hbm, gj_hbm,
         y_v, idx_v, acc_sh, acc_lo, g_v, sem):
    c = lax.axis_index("c")
    s = lax.axis_index("s")

    @pl.when(s == 0)
    def _zero():
        @pl.loop(0, N_NODES // 16)
        def _(i):
            acc_lo[pl.ds(i * 16, 16)] = jnp.zeros((16,), jnp.float32)
        pltpu.async_copy(acc_lo, acc_sh, sem).wait()

    pltpu.async_copy(y_hbm.at[s], y_v, sem).wait()

    @pl.when(c == 0)
    def _():
        pltpu.async_copy(src_hbm.at[s], idx_v, sem).wait()

    @pl.when(c == 1)
    def _():
        pltpu.async_copy(dst_hbm.at[s], idx_v, sem).wait()

    plsc.subcore_barrier()

    @pl.loop(0, ROWS_PER_SUB)
    def _scatter(j):
        pltpu.sync_copy(y_v.at[j], acc_sh.at[idx_v.at[j]], add=True)

    plsc.subcore_barrier()
    pltpu.async_copy(acc_sh, acc_lo, sem).wait()

    @pl.loop(0, ROWS_PER_SUB)
    def _gatherback(j):
        for k in range(W_G // 16):
            idx16 = idx_v[j, pl.ds(k * 16, 16)]
            g_v[j, pl.ds(k * 16, 16)] = plsc.load_gather(acc_lo, [idx16])

    @pl.when(c == 0)
    def _():
        pltpu.async_copy(g_v, gi_hbm.at[s], sem).wait()

    @pl.when(c == 1)
    def _():
        pltpu.async_copy(g_v, gj_hbm.at[s], sem).wait()


def _coef_body(y_ref, gi_ref, xs_ref, gj_ref, xt_ref,
               fw0_ref, fb0_ref, fg_ref, fb_ref, fw1_ref, fb1_ref,
               out_ref):
    vs = (y_ref[...], gi_ref[...], xs_ref[...], gj_ref[...], xt_ref[...])
    fw0 = fw0_ref[...]
    h = vs[0][:, None, :] * fw0[:, 0:1][None]
    for k in range(1, 5):
        h = h + vs[k][:, None, :] * fw0[:, k:k + 1][None]
    h = _leaky(h + fb0_ref[...][None])
    m = jnp.mean(h, axis=1, keepdims=True)
    v = jnp.mean(h * h, axis=1, keepdims=True) - m * m
    hn = (h - m) * lax.rsqrt(v + 1e-5) * fg_ref[...][None] + fb_ref[...][None]
    cf = jnp.sum(hn * fw1_ref[...][None], axis=1) + fb1_ref[...]
    out_ref[...] = vs[0] * jnp.maximum(cf, 0.0)


def _coef(y_r, gi_r, xs_r, gj_r, xt_r, fw0, fb0, fg, fb, fw1, fb1):
    rows = E_TOTAL // C_COEF
    nb = rows // BR
    rspec = lambda: pl.BlockSpec((BR, C_COEF), lambda i: (i, 0))
    wspec = lambda r, c: pl.BlockSpec((r, c), lambda i: (0, 0))
    return pl.pallas_call(
        _coef_body,
        grid=(nb,),
        in_specs=[rspec(), rspec(), rspec(), rspec(), rspec(),
                  wspec(64, 5), wspec(64, 1), wspec(64, 1), wspec(64, 1),
                  wspec(64, 1), wspec(1, 1)],
        out_specs=rspec(),
        out_shape=jax.ShapeDtypeStruct((rows, C_COEF), jnp.float32),
        compiler_params=pltpu.CompilerParams(
            dimension_semantics=("parallel",)),
    )(y_r, gi_r, xs_r, gj_r, xt_r, fw0, fb0, fg, fb, fw1, fb1)


def kernel(emb_s, emb_t, at_s, at_t, x_s, x_t, edge_index, edge_weight,
           W0, b0, W1, b1, W2, b2, ln0_g, ln0_b, ln1_g, ln1_b,
           fW0, fb0, fln_g, fln_b, fW1, fb1):
    src32 = edge_index[0].reshape(N_TILES, ROWS_PER_TILE, W_G)
    dst32 = edge_index[1].reshape(N_TILES, ROWS_PER_TILE, W_G)
    src16 = edge_index[0].reshape(16, ROWS_PER_SUB, W_G)
    dst16 = edge_index[1].reshape(16, ROWS_PER_SUB, W_G)
    tabs, ns_s = _pre(emb_s, W0[:, 0:D].T)
    tabt, ns_t = _pre(emb_t, W0[:, D:2 * D].T)
    gs, gt = _gather(tabs, tabt, src32, dst32)
    nsum, xs_e, xt_e = _scal(ns_s, ns_t, x_s[:, 1], x_t[:, 1], src32, dst32)
    row = lambda a: a.reshape(1, -1)
    ecol = lambda a: a.reshape(E_TOTAL, 1)
    y = _fnn(
        gs, gt, ecol(nsum), ecol(xs_e), ecol(xt_e),
        at_s[1], at_t[1], edge_weight,
        W0[:, 2 * D:].T, row(b0), row(ln0_g), row(ln0_b),
        W1.T, row(b1), row(ln1_g), row(ln1_b),
        row(W2[0]), b2.reshape(1, 1))
    rc = E_TOTAL // C_COEF
    xs_r = xs_e.reshape(rc, C_COEF)
    xt_r = xt_e.reshape(rc, C_COEF)
    fargs = (fW0, fb0.reshape(64, 1), fln_g.reshape(64, 1),
             fln_b.reshape(64, 1), fW1.reshape(64, 1), fb1.reshape(1, 1))
    y = y.reshape(E_TOTAL)
    for _ in range(2):
        gi2, gj2 = _seg(y.reshape(16, ROWS_PER_SUB, W_G), src16, dst16)
        y = _coef(y.reshape(rc, C_COEF),
                  gi2.reshape(rc, C_COEF), xs_r,
                  gj2.reshape(rc, C_COEF), xt_r, *fargs)
        y = y.reshape(E_TOTAL)
    return y


# speedup vs baseline: 7.5538x; 1.0272x over previous
import dataclasses
import functools

import jax
import jax.numpy as jnp
from jax import lax
from jax.experimental import pallas as pl
from jax.experimental.pallas import tpu as pltpu
from jax.experimental.pallas import tpu_sc as plsc

N_NODES = 10000
E_TOTAL = 320000
D = 128
W_G = 80
ROWS = E_TOTAL // W_G
N_TILES = 32
ROWS_PER_TILE = ROWS // N_TILES
ROWS_PER_SUB = ROWS // 16
BE = 2560
C_COEF = 160
BR = 16

_mesh = plsc.VectorSubcoreMesh(core_axis_name="c", subcore_axis_name="s")

_sc_cp = pltpu.CompilerParams()
if "needs_layout_passes" in pltpu.CompilerParams.__dataclass_fields__:
    _sc_cp = dataclasses.replace(_sc_cp, needs_layout_passes=False)


def _leaky(x):
    return jnp.where(x >= 0, x, 0.01 * x)


def _ln(x, g, b):
    m = jnp.mean(x, axis=-1, keepdims=True)
    v = jnp.mean((x - m) ** 2, axis=-1, keepdims=True)
    return (x - m) * lax.rsqrt(v + 1e-5) * g + b


def _pre_body(emb_ref, w_ref, tab_ref, ns_ref):
    e = emb_ref[...]
    tab_ref[...] = jnp.dot(e, w_ref[...], precision=lax.Precision.HIGHEST,
                           preferred_element_type=jnp.float32)
    ns_ref[...] = jnp.sum(e * e, axis=1)


def _pre(emb, w0part):
    n = emb.shape[0]
    return pl.pallas_call(
        _pre_body,
        out_shape=(jax.ShapeDtypeStruct((n, D), jnp.float32),
                   jax.ShapeDtypeStruct((n,), jnp.float32)),
    )(emb, w0part)


ROWS_CT = E_TOTAL // 16 // W_G
N_PH = 5
PR = ROWS_CT // N_PH


@functools.partial(
    pl.kernel, mesh=_mesh,
    out_type=(jax.ShapeDtypeStruct((E_TOTAL, D), jnp.float32),
              jax.ShapeDtypeStruct((E_TOTAL, D), jnp.float32)),
    scratch_types=[
        pltpu.VMEM_SHARED((N_NODES, D), jnp.float32),
        pltpu.VMEM((PR, W_G), jnp.int32),
        pltpu.VMEM((W_G, D), jnp.float32),
        pltpu.VMEM((W_G, D), jnp.float32),
        pltpu.SemaphoreType.DMA,
        pltpu.SemaphoreType.DMA,
        pltpu.SemaphoreType.DMA,
        pltpu.SemaphoreType.DMA,
        pltpu.SemaphoreType.DMA,
    ],
    compiler_params=_sc_cp,
)
def _gather(tabs_hbm, tabt_hbm, src_hbm, dst_hbm,
            gs_hbm, gt_hbm,
            spm_tab, idx_v, buf_a, buf_b, ga, gb, oa, ob, misc):
    c = lax.axis_index("c")
    s = lax.axis_index("s")

    @pl.when(c == 0)
    def _():
        @pl.when(s == 0)
        def _():
            pltpu.async_copy(tabs_hbm, spm_tab, misc).wait()

    @pl.when(c == 1)
    def _():
        @pl.when(s == 0)
        def _():
            pltpu.async_copy(tabt_hbm, spm_tab, misc).wait()

    plsc.subcore_barrier()

    def run_side(idx_hbm, out_hbm):
        def g_start(j, buf, sem):
            pltpu.make_async_copy(spm_tab.at[idx_v.at[j]], buf, sem).start()

        def g_wait(j, buf, sem):
            pltpu.make_async_copy(spm_tab.at[idx_v.at[j]], buf, sem).wait()

        def o_start(p, j, buf, sem):
            e0 = (s * ROWS_CT + p * PR + j) * W_G
            pltpu.make_async_copy(buf, out_hbm.at[pl.ds(e0, W_G)],
                                  sem).start()

        def o_wait(p, j, buf, sem):
            e0 = (s * ROWS_CT + p * PR + j) * W_G
            pltpu.make_async_copy(buf, out_hbm.at[pl.ds(e0, W_G)],
                                  sem).wait()

        @pl.loop(0, N_PH)
        def _phase(p):
            pltpu.async_copy(idx_hbm.at[s].at[p], idx_v, misc).wait()
            g_start(0, buf_a, ga)
            g_start(1, buf_b, gb)

            @pl.loop(0, PR // 2 - 1)
            def _(i):
                j0 = 2 * i
                j1 = j0 + 1
                g_wait(j0, buf_a, ga)
                o_start(p, j0, buf_a, oa)
                g_wait(j1, buf_b, gb)
                o_start(p, j1, buf_b, ob)
                o_wait(p, j0, buf_a, oa)
                g_start(j0 + 2, buf_a, ga)
                o_wait(p, j1, buf_b, ob)
                g_start(j1 + 2, buf_b, gb)

            j0 = PR - 2
            j1 = PR - 1
            g_wait(j0, buf_a, ga)
            o_start(p, j0, buf_a, oa)
            g_wait(j1, buf_b, gb)
            o_start(p, j1, buf_b, ob)
            o_wait(p, j0, buf_a, oa)
            o_wait(p, j1, buf_b, ob)

    @pl.when(c == 0)
    def _():
        run_side(src_hbm, gs_hbm)

    @pl.when(c == 1)
    def _():
        run_side(dst_hbm, gt_hbm)


@functools.partial(
    pl.kernel, mesh=_mesh,
    out_type=(jax.ShapeDtypeStruct((N_TILES, ROWS_PER_TILE, W_G), jnp.float32),
              jax.ShapeDtypeStruct((N_TILES, ROWS_PER_TILE, W_G), jnp.float32),
              jax.ShapeDtypeStruct((N_TILES, ROWS_PER_TILE, W_G), jnp.float32)),
    scratch_types=[
        pltpu.VMEM((ROWS_PER_TILE, W_G), jnp.int32),
        pltpu.VMEM((ROWS_PER_TILE, W_G), jnp.int32),
        pltpu.VMEM((N_NODES,), jnp.float32),
        pltpu.VMEM((N_NODES,), jnp.float32),
        pltpu.VMEM((N_NODES,), jnp.float32),
        pltpu.VMEM((N_NODES,), jnp.float32),
        pltpu.VMEM((ROWS_PER_TILE, W_G), jnp.float32),
        pltpu.VMEM((ROWS_PER_TILE, W_G), jnp.float32),
        pltpu.VMEM((ROWS_PER_TILE, W_G), jnp.float32),
        pltpu.SemaphoreType.DMA,
    ],
    compiler_params=_sc_cp,
)
def _scal(nss_hbm, nst_hbm, x1s_hbm, x1t_hbm, src_hbm, dst_hbm,
          nsum_hbm, xs1_hbm, xt1_hbm,
          idxs_v, idxt_v, nss_v, nst_v, x1s_v, x1t_v,
          nsum_v, xs1_v, xt1_v, sem):
    wid = lax.axis_index("s") * 2 + lax.axis_index("c")
    pltpu.async_copy(src_hbm.at[wid], idxs_v, sem).wait()
    pltpu.async_copy(dst_hbm.at[wid], idxt_v, sem).wait()
    pltpu.async_copy(nss_hbm, nss_v, sem).wait()
    pltpu.async_copy(nst_hbm, nst_v, sem).wait()
    pltpu.async_copy(x1s_hbm, x1s_v, sem).wait()
    pltpu.async_copy(x1t_hbm, x1t_v, sem).wait()

    @pl.loop(0, ROWS_PER_TILE)
    def _row(j):
        for k in range(W_G // 16):
            sl = pl.ds(k * 16, 16)
            i16s = idxs_v[j, sl]
            i16t = idxt_v[j, sl]
            nsum_v[j, sl] = (plsc.load_gather(nss_v, [i16s])
                             + plsc.load_gather(nst_v, [i16t]))
            xs1_v[j, sl] = plsc.load_gather(x1s_v, [i16s])
            xt1_v[j, sl] = plsc.load_gather(x1t_v, [i16t])

    pltpu.async_copy(nsum_v, nsum_hbm.at[wid], sem).wait()
    pltpu.async_copy(xs1_v, xs1_hbm.at[wid], sem).wait()
    pltpu.async_copy(xt1_v, xt1_hbm.at[wid], sem).wait()


def _fnn_body(gs_ref, gt_ref, nss_ref, xs1_ref, xt1_ref,
              ats_ref, att_ref, ew_ref,
              w0x_ref, b0_ref, g0_ref, bb0_ref,
              w1t_ref, b1_ref, g1_ref, bb1_ref,
              w2_ref, b2_ref,
              y_ref):
    ps = gs_ref[...] + gt_ref[...]
    nsum = nss_ref[...]
    xs1 = xs1_ref[...]
    xt1 = xt1_ref[...]
    asm = jnp.mean(ats_ref[...], axis=1, keepdims=True)
    atm = jnp.mean(att_ref[...], axis=1, keepdims=True)
    ew = ew_ref[...]
    w0x = w0x_ref[...]
    z = (ps + xs1 * w0x[0:1, :] + xt1 * w0x[1:2, :] + asm * w0x[2:3, :]
         + atm * w0x[3:4, :] + ew * w0x[4:5, :])
    n2 = nsum + xs1 * xs1 + xt1 * xt1 + asm * asm + atm * atm + ew * ew
    inv = 1.0 / jnp.maximum(jnp.sqrt(n2), 1e-12)
    z = z * inv + b0_ref[...]
    y1 = _ln(_leaky(z), g0_ref[...], bb0_ref[...])
    z2 = jnp.dot(y1, w1t_ref[...], precision=lax.Precision.HIGHEST,
                 preferred_element_type=jnp.float32)
    y2 = _ln(_leaky(z2 + b1_ref[...]), g1_ref[...], bb1_ref[...])
    yv = jnp.sum(y2 * w2_ref[...], axis=1, keepdims=True) + b2_ref[...]
    y_ref[...] = jnp.maximum(yv, 0.0)


def _fnn(gs, gt, nss_e, xs1e, xt1e, ats1, att1, ew,
         w0x, b0, g0, bb0, w1t, b1, g1, bb1, w2, b2):
    nb = E_TOTAL // BE
    col = lambda: pl.BlockSpec((BE, 1), lambda i: (i, 0))
    wspec = lambda r, c: pl.BlockSpec((r, c), lambda i: (0, 0))
    return pl.pallas_call(
        _fnn_body,
        grid=(nb,),
        in_specs=[
            pl.BlockSpec((BE, D), lambda i: (i, 0)),
            pl.BlockSpec((BE, D), lambda i: (i, 0)),
            col(), col(), col(),
            pl.BlockSpec((BE, 8), lambda i: (i, 0)),
            pl.BlockSpec((BE, 8), lambda i: (i, 0)),
            col(),
            wspec(5, D), wspec(1, D), wspec(1, D), wspec(1, D),
            wspec(D, D), wspec(1, D), wspec(1, D), wspec(1, D),
            wspec(1, D), wspec(1, 1),
        ],
        out_specs=col(),
        out_shape=jax.ShapeDtypeStruct((E_TOTAL, 1), jnp.float32),
        compiler_params=pltpu.CompilerParams(
            dimension_semantics=("parallel",)),
    )(gs, gt, nss_e, xs1e, xt1e, ats1, att1, ew,
      w0x, b0, g0, bb0, w1t, b1, g1, bb1, w2, b2)


@functools.partial(
    pl.kernel, mesh=_mesh,
    out_type=(jax.ShapeDtypeStruct((16, ROWS_PER_SUB, W_G), jnp.float32),
              jax.ShapeDtypeStruct((16, ROWS_PER_SUB, W_G), jnp.float32)),
    scratch_types=[
        pltpu.VMEM((ROWS_PER_SUB, W_G), jnp.float32),
        pltpu.VMEM((ROWS_PER_SUB, W_G), jnp.int32),
        pltpu.VMEM_SHARED((N_NODES,), jnp.float32),
        pltpu.VMEM((N_NODES,), jnp.float32),
        pltpu.VMEM((ROWS_PER_SUB, W_G), jnp.float32),
        pltpu.SemaphoreType.DMA,
    ],
    compiler_params=_sc_cp,
)
def _seg(y_hbm, src_hbm, dst_hbm, gi_hbm, gj_hbm,
         y_v, idx_v, acc_sh, acc_lo, g_v, sem):
    c = lax.axis_index("c")
    s = lax.axis_index("s")

    @pl.when(s == 0)
    def _zero():
        @pl.loop(0, N_NODES // 16)
        def _(i):
            acc_lo[pl.ds(i * 16, 16)] = jnp.zeros((16,), jnp.float32)
        pltpu.async_copy(acc_lo, acc_sh, sem).wait()

    pltpu.async_copy(y_hbm.at[s], y_v, sem).wait()

    @pl.when(c == 0)
    def _():
        pltpu.async_copy(src_hbm.at[s], idx_v, sem).wait()

    @pl.when(c == 1)
    def _():
        pltpu.async_copy(dst_hbm.at[s], idx_v, sem).wait()

    plsc.subcore_barrier()

    @pl.loop(0, ROWS_PER_SUB)
    def _scatter(j):
        pltpu.sync_copy(y_v.at[j], acc_sh.at[idx_v.at[j]], add=True)

    plsc.subcore_barrier()
    pltpu.async_copy(acc_sh, acc_lo, sem).wait()

    @pl.loop(0, ROWS_PER_SUB)
    def _gatherback(j):
        for k in range(W_G // 16):
            idx16 = idx_v[j, pl.ds(k * 16, 16)]
            g_v[j, pl.ds(k * 16, 16)] = plsc.load_gather(acc_lo, [idx16])

    @pl.when(c == 0)
    def _():
        pltpu.async_copy(g_v, gi_hbm.at[s], sem).wait()

    @pl.when(c == 1)
    def _():
        pltpu.async_copy(g_v, gj_hbm.at[s], sem).wait()


def _coef_body(y_ref, gi_ref, xs_ref, gj_ref, xt_ref,
               fw0_ref, fb0_ref, fg_ref, fb_ref, fw1_ref, fb1_ref,
               out_ref):
    vs = (y_ref[...], gi_ref[...], xs_ref[...], gj_ref[...], xt_ref[...])
    fw0 = fw0_ref[...]
    h = vs[0][:, None, :] * fw0[:, 0:1][None]
    for k in range(1, 5):
        h = h + vs[k][:, None, :] * fw0[:, k:k + 1][None]
    h = _leaky(h + fb0_ref[...][None])
    m = jnp.mean(h, axis=1, keepdims=True)
    v = jnp.mean(h * h, axis=1, keepdims=True) - m * m
    hn = (h - m) * lax.rsqrt(v + 1e-5) * fg_ref[...][None] + fb_ref[...][None]
    cf = jnp.sum(hn * fw1_ref[...][None], axis=1) + fb1_ref[...]
    out_ref[...] = vs[0] * jnp.maximum(cf, 0.0)


def _coef(y_r, gi_r, xs_r, gj_r, xt_r, fw0, fb0, fg, fb, fw1, fb1):
    rows = E_TOTAL // C_COEF
    nb = rows // BR
    rspec = lambda: pl.BlockSpec((BR, C_COEF), lambda i: (i, 0))
    wspec = lambda r, c: pl.BlockSpec((r, c), lambda i: (0, 0))
    return pl.pallas_call(
        _coef_body,
        grid=(nb,),
        in_specs=[rspec(), rspec(), rspec(), rspec(), rspec(),
                  wspec(64, 5), wspec(64, 1), wspec(64, 1), wspec(64, 1),
                  wspec(64, 1), wspec(1, 1)],
        out_specs=rspec(),
        out_shape=jax.ShapeDtypeStruct((rows, C_COEF), jnp.float32),
        compiler_params=pltpu.CompilerParams(
            dimension_semantics=("parallel",)),
    )(y_r, gi_r, xs_r, gj_r, xt_r, fw0, fb0, fg, fb, fw1, fb1)


def kernel(emb_s, emb_t, at_s, at_t, x_s, x_t, edge_index, edge_weight,
           W0, b0, W1, b1, W2, b2, ln0_g, ln0_b, ln1_g, ln1_b,
           fW0, fb0, fln_g, fln_b, fW1, fb1):
    src16 = edge_index[0].reshape(16, ROWS_PER_SUB, W_G)
    dst16 = edge_index[1].reshape(16, ROWS_PER_SUB, W_G)
    tabs, ns_s = _pre(emb_s, W0[:, 0:D].T)
    tabt, ns_t = _pre(emb_t, W0[:, D:2 * D].T)
    src32 = edge_index[0].reshape(N_TILES, ROWS_PER_TILE, W_G)
    dst32 = edge_index[1].reshape(N_TILES, ROWS_PER_TILE, W_G)
    srcp = edge_index[0].reshape(16, N_PH, PR, W_G)
    dstp = edge_index[1].reshape(16, N_PH, PR, W_G)
    gs, gt = _gather(tabs, tabt, srcp, dstp)
    nsum, xs_e, xt_e = _scal(ns_s, ns_t, x_s[:, 1], x_t[:, 1], src32, dst32)
    row = lambda a: a.reshape(1, -1)
    ecol = lambda a: a.reshape(E_TOTAL, 1)
    y = _fnn(
        gs, gt, ecol(nsum), ecol(xs_e), ecol(xt_e),
        at_s[1], at_t[1], edge_weight,
        W0[:, 2 * D:].T, row(b0), row(ln0_g), row(ln0_b),
        W1.T, row(b1), row(ln1_g), row(ln1_b),
        row(W2[0]), b2.reshape(1, 1))
    rc = E_TOTAL // C_COEF
    xs_r = xs_e.reshape(rc, C_COEF)
    xt_r = xt_e.reshape(rc, C_COEF)
    fargs = (fW0, fb0.reshape(64, 1), fln_g.reshape(64, 1),
             fln_b.reshape(64, 1), fW1.reshape(64, 1), fb1.reshape(1, 1))
    y = y.reshape(E_TOTAL)
    for _ in range(2):
        gi2, gj2 = _seg(y.reshape(16, ROWS_PER_SUB, W_G), src16, dst16)
        y = _coef(y.reshape(rc, C_COEF),
                  gi2.reshape(rc, C_COEF), xs_r,
                  gj2.reshape(rc, C_COEF), xt_r, *fargs)
        y = y.reshape(E_TOTAL)
    return y


# speedup vs baseline: 14.4317x; 1.9105x over previous
import dataclasses
import functools

import jax
import jax.numpy as jnp
from jax import lax
from jax.experimental import pallas as pl
from jax.experimental.pallas import tpu as pltpu
from jax.experimental.pallas import tpu_sc as plsc

N_NODES = 10000
E_TOTAL = 320000
D = 128
PE = 327680
PROWS = PE // D
RT = PROWS // 16
N_PH = 5
PR = RT // N_PH
BE = 4096
NB = PE // BE
CH = BE // D
HI = lax.Precision.HIGHEST

_mesh = plsc.VectorSubcoreMesh(core_axis_name="c", subcore_axis_name="s")

_sc_cp = pltpu.CompilerParams()
if "needs_layout_passes" in pltpu.CompilerParams.__dataclass_fields__:
    _sc_cp = dataclasses.replace(_sc_cp, needs_layout_passes=False)


def _leaky(x):
    return jnp.where(x >= 0, x, 0.01 * x)


def _lnT(x, g, b):
    m = jnp.mean(x, axis=0, keepdims=True)
    xc = x - m
    v = jnp.mean(xc * xc, axis=0, keepdims=True)
    return xc * (1.0 / jnp.sqrt(v + 1e-5)) * g + b


def _pre_body(embs_ref, embt_ref, nss_ref, nst_ref):
    es = embs_ref[...]
    et = embt_ref[...]
    nss_ref[...] = jnp.sum(es * es, axis=1)
    nst_ref[...] = jnp.sum(et * et, axis=1)


def _pre(emb_s, emb_t):
    n = emb_s.shape[0]
    return pl.pallas_call(
        _pre_body,
        out_shape=(jax.ShapeDtypeStruct((n,), jnp.float32),
                   jax.ShapeDtypeStruct((n,), jnp.float32)),
    )(emb_s, emb_t)


@functools.partial(
    pl.kernel, mesh=_mesh,
    out_type=(jax.ShapeDtypeStruct((PE, D), jnp.float32),
              jax.ShapeDtypeStruct((PE, D), jnp.float32)),
    scratch_types=[
        pltpu.VMEM_SHARED((N_NODES, D), jnp.float32),
        pltpu.VMEM((PR, D), jnp.int32),
        pltpu.VMEM((D, D), jnp.float32),
        pltpu.VMEM((D, D), jnp.float32),
        pltpu.SemaphoreType.DMA,
        pltpu.SemaphoreType.DMA,
        pltpu.SemaphoreType.DMA,
        pltpu.SemaphoreType.DMA,
        pltpu.SemaphoreType.DMA,
    ],
    compiler_params=_sc_cp,
)
def _gather(tabs_hbm, tabt_hbm, src_hbm, dst_hbm,
            gs_hbm, gt_hbm,
            spm_tab, idx_v, buf_a, buf_b, ga, gb, oa, ob, misc):
    c = lax.axis_index("c")
    s = lax.axis_index("s")

    @pl.when(c == 0)
    def _():
        @pl.when(s == 0)
        def _():
            pltpu.async_copy(tabs_hbm, spm_tab, misc).wait()

    @pl.when(c == 1)
    def _():
        @pl.when(s == 0)
        def _():
            pltpu.async_copy(tabt_hbm, spm_tab, misc).wait()

    plsc.subcore_barrier()

    def run_side(idx_hbm, out_hbm):
        def g_start(j, buf, sem):
            pltpu.make_async_copy(spm_tab.at[idx_v.at[j]], buf, sem).start()

        def g_wait(j, buf, sem):
            pltpu.make_async_copy(spm_tab.at[idx_v.at[j]], buf, sem).wait()

        def o_copy(p, j, buf, sem):
            e0 = (s * RT + p * PR + j) * D
            return pltpu.make_async_copy(buf, out_hbm.at[pl.ds(e0, D)], sem)

        @pl.loop(0, N_PH)
        def _phase(p):
            pltpu.async_copy(idx_hbm.at[s].at[p], idx_v, misc).wait()
            g_start(0, buf_a, ga)
            g_start(1, buf_b, gb)

            @pl.loop(0, PR // 2 - 1)
            def _(i):
                j0 = 2 * i
                j1 = j0 + 1
                g_wait(j0, buf_a, ga)
                o_copy(p, j0, buf_a, oa).start()
                g_wait(j1, buf_b, gb)
                o_copy(p, j1, buf_b, ob).start()
                o_copy(p, j0, buf_a, oa).wait()
                g_start(j0 + 2, buf_a, ga)
                o_copy(p, j1, buf_b, ob).wait()
                g_start(j1 + 2, buf_b, gb)

            j0 = PR - 2
            j1 = PR - 1
            g_wait(j0, buf_a, ga)
            o_copy(p, j0, buf_a, oa).start()
            g_wait(j1, buf_b, gb)
            o_copy(p, j1, buf_b, ob).start()
            o_copy(p, j0, buf_a, oa).wait()
            o_copy(p, j1, buf_b, ob).wait()

    @pl.when(c == 0)
    def _():
        run_side(src_hbm, gs_hbm)

    @pl.when(c == 1)
    def _():
        run_side(dst_hbm, gt_hbm)


@functools.partial(
    pl.kernel, mesh=_mesh,
    out_type=(jax.ShapeDtypeStruct((16, RT, D), jnp.float32),
              jax.ShapeDtypeStruct((16, RT, D), jnp.float32),
              jax.ShapeDtypeStruct((16, RT, D), jnp.float32),
              jax.ShapeDtypeStruct((16, RT, D), jnp.float32)),
    scratch_types=[
        pltpu.VMEM((RT, D), jnp.int32),
        pltpu.VMEM((N_NODES,), jnp.float32),
        pltpu.VMEM((N_NODES,), jnp.float32),
        pltpu.VMEM((RT, D), jnp.float32),
        pltpu.VMEM((RT, D), jnp.float32),
        pltpu.SemaphoreType.DMA,
    ],
    compiler_params=_sc_cp,
)
def _scal(nss_hbm, nst_hbm, x1s_hbm, x1t_hbm, src_hbm, dst_hbm,
          nse_s_hbm, x1e_s_hbm, nse_t_hbm, x1e_t_hbm,
          idx_v, ns_tab, x1_tab, ns_buf, x1_buf, sem):
    c = lax.axis_index("c")
    s = lax.axis_index("s")

    @pl.when(c == 0)
    def _():
        pltpu.async_copy(nss_hbm, ns_tab, sem).wait()
        pltpu.async_copy(x1s_hbm, x1_tab, sem).wait()
        pltpu.async_copy(src_hbm.at[s], idx_v, sem).wait()

    @pl.when(c == 1)
    def _():
        pltpu.async_copy(nst_hbm, ns_tab, sem).wait()
        pltpu.async_copy(x1t_hbm, x1_tab, sem).wait()
        pltpu.async_copy(dst_hbm.at[s], idx_v, sem).wait()

    @pl.loop(0, RT)
    def _row(j):
        for k in range(D // 16):
            sl = pl.ds(k * 16, 16)
            i16 = idx_v[j, sl]
            ns_buf[j, sl] = plsc.load_gather(ns_tab, [i16])
            x1_buf[j, sl] = plsc.load_gather(x1_tab, [i16])

    @pl.when(c == 0)
    def _():
        pltpu.async_copy(ns_buf, nse_s_hbm.at[s], sem).wait()
        pltpu.async_copy(x1_buf, x1e_s_hbm.at[s], sem).wait()

    @pl.when(c == 1)
    def _():
        pltpu.async_copy(ns_buf, nse_t_hbm.at[s], sem).wait()
        pltpu.async_copy(x1_buf, x1e_t_hbm.at[s], sem).wait()


def _bf(x):
    return x.astype(jnp.bfloat16)


def _fnn_body(gs_ref, gt_ref, nss_ref, nst_ref, xs1_ref, xt1_ref,
              ats_ref, att_ref, ew_ref, m8_ref,
              w0a_ref, w0b_ref, w0xc_ref, b0c_ref, g0c_ref, bb0c_ref,
              w1_ref, b1c_ref, g1c_ref, bb1c_ref,
              w2r_ref, b2_ref,
              y_ref):
    i = pl.program_id(0)
    m8 = m8_ref[...]
    asm = jnp.dot(ats_ref[...], m8, precision=HI,
                  preferred_element_type=jnp.float32) * 0.125
    atm = jnp.dot(att_ref[...], m8, precision=HI,
                  preferred_element_type=jnp.float32) * 0.125
    ew = ew_ref[...]
    xs1 = xs1_ref[...]
    xt1 = xt1_ref[...]
    n2 = (nss_ref[...] + nst_ref[...] + xs1 * xs1 + xt1 * xt1
          + asm * asm + atm * atm + ew * ew)
    inv = 1.0 / jnp.maximum(jnp.sqrt(n2), 1e-12)
    w0xc = w0xc_ref[...]
    es_chunks = []
    et_chunks = []
    zx_chunks = []
    for k in range(CH):
        sl = slice(k * D, (k + 1) * D)
        ik = inv[k:k + 1, :]
        es_chunks.append(lax.transpose(gs_ref[sl, :], (1, 0)) * ik)
        et_chunks.append(lax.transpose(gt_ref[sl, :], (1, 0)) * ik)
        zx = jnp.zeros((D, D), jnp.float32)
        for j, sc in enumerate((xs1, xt1, asm, atm, ew)):
            scn = _bf(sc[k:k + 1, :] * ik).astype(jnp.float32)
            zx = zx + w0xc[:, j:j + 1].astype(jnp.float32) * scn
        zx_chunks.append(zx)
    esn = _bf(jnp.concatenate(es_chunks, axis=1))
    etn = _bf(jnp.concatenate(et_chunks, axis=1))
    zx = jnp.concatenate(zx_chunks, axis=1)
    cdims = (((1,), (0,)), ((), ()))
    z = (lax.dot_general(w0a_ref[...], esn, cdims,
                         preferred_element_type=jnp.float32)
         + lax.dot_general(w0b_ref[...], etn, cdims,
                           preferred_element_type=jnp.float32)
         + zx + b0c_ref[...])
    y1 = _lnT(_leaky(z), g0c_ref[...], bb0c_ref[...])
    z2 = lax.dot_general(w1_ref[...], _bf(y1), cdims,
                         preferred_element_type=jnp.float32)
    y2 = _lnT(_leaky(z2 + b1c_ref[...]), g1c_ref[...], bb1c_ref[...])
    yv = lax.dot_general(w2r_ref[...], _bf(y2), cdims,
                         preferred_element_type=jnp.float32) + b2_ref[...]
    yv = jnp.maximum(yv, 0.0)
    live_rows = E_TOTAL // D
    for k in range(CH):
        sl = slice(k * D, (k + 1) * D)
        live = (CH * i + k) < live_rows
        y_ref[k:k + 1, :] = jnp.where(live, yv[:, sl], 0.0)


def _fnn(gs, gt, nss_e, nst_e, xs1e, xt1e, ats_p, att_p, ew_p, m8,
         w0a, w0b, w0xc, b0c, g0c, bb0c, w1, b1c, g1c, bb1c, w2r, b2):
    rsp = lambda: pl.BlockSpec((CH, D), lambda i: (i, 0))
    wspec = lambda r, c: pl.BlockSpec((r, c), lambda i: (0, 0))
    return pl.pallas_call(
        _fnn_body,
        grid=(NB,),
        in_specs=[
            pl.BlockSpec((BE, D), lambda i: (i, 0)),
            pl.BlockSpec((BE, D), lambda i: (i, 0)),
            rsp(), rsp(), rsp(), rsp(),
            pl.BlockSpec((CH, 1024), lambda i: (i, 0)),
            pl.BlockSpec((CH, 1024), lambda i: (i, 0)),
            rsp(),
            wspec(1024, D),
            wspec(D, D), wspec(D, D),
            wspec(D, 5), wspec(D, 1), wspec(D, 1), wspec(D, 1),
            wspec(D, D), wspec(D, 1), wspec(D, 1), wspec(D, 1),
            wspec(1, D), wspec(1, 1),
        ],
        out_specs=rsp(),
        out_shape=jax.ShapeDtypeStruct((PROWS, D), jnp.float32),
        compiler_params=pltpu.CompilerParams(
            dimension_semantics=("parallel",)),
    )(gs, gt, nss_e, nst_e, xs1e, xt1e, ats_p, att_p, ew_p, m8,
      w0a, w0b, w0xc, b0c, g0c, bb0c, w1, b1c, g1c, bb1c, w2r, b2)


@functools.partial(
    pl.kernel, mesh=_mesh,
    out_type=(jax.ShapeDtypeStruct((16, RT, D), jnp.float32),
              jax.ShapeDtypeStruct((16, RT, D), jnp.float32)),
    scratch_types=[
        pltpu.VMEM((RT, D), jnp.float32),
        pltpu.VMEM((RT, D), jnp.int32),
        pltpu.VMEM_SHARED((N_NODES,), jnp.float32),
        pltpu.VMEM((N_NODES,), jnp.float32),
        pltpu.VMEM((RT, D), jnp.float32),
        pltpu.SemaphoreType.DMA,
    ],
    compiler_params=_sc_cp,
)
def _seg(y_hbm, src_hbm, dst_hbm, gi_hbm, gj_hbm,
         y_v, idx_v, acc_sh, acc_lo, g_v, sem):
    c = lax.axis_index("c")
    s = lax.axis_index("s")

    @pl.when(s == 0)
    def _zero():
        @pl.loop(0, N_NODES // 16)
        def _(i):
            acc_lo[pl.ds(i * 16, 16)] = jnp.zeros((16,), jnp.float32)
        pltpu.async_copy(acc_lo, acc_sh, sem).wait()

    pltpu.async_copy(y_hbm.at[s], y_v, sem).wait()

    @pl.when(c == 0)
    def _():
        pltpu.async_copy(src_hbm.at[s], idx_v, sem).wait()

    @pl.when(c == 1)
    def _():
        pltpu.async_copy(dst_hbm.at[s], idx_v, sem).wait()

    plsc.subcore_barrier()

    @pl.loop(0, RT)
    def _scatter(j):
        pltpu.sync_copy(y_v.at[j], acc_sh.at[idx_v.at[j]], add=True)

    plsc.subcore_barrier()
    pltpu.async_copy(acc_sh, acc_lo, sem).wait()

    @pl.loop(0, RT)
    def _gatherback(j):
        for k in range(D // 16):
            sl = pl.ds(k * 16, 16)
            g_v[j, sl] = plsc.load_gather(acc_lo, [idx_v[j, sl]])

    @pl.when(c == 0)
    def _():
        pltpu.async_copy(g_v, gi_hbm.at[s], sem).wait()

    @pl.when(c == 1)
    def _():
        pltpu.async_copy(g_v, gj_hbm.at[s], sem).wait()


CW = 1024
CROWS = PE // CW
CB = 8


def _coef_body(y_ref, gi_ref, xs_ref, gj_ref, xt_ref,
               fw0_ref, fb0c_ref, fg_ref, fb_ref, fw1r_ref, fb1_ref,
               out_ref):
    fw0 = fw0_ref[...]
    fb0c = fb0c_ref[...]
    fg = fg_ref[...]
    fb = fb_ref[...]
    fw1r = fw1r_ref[...]
    fb1 = fb1_ref[...]
    cdims = (((1,), (0,)), ((), ()))
    for r in range(CB):
        rs = slice(r, r + 1)
        v = jnp.concatenate([y_ref[rs, :], gi_ref[rs, :], xs_ref[rs, :],
                             gj_ref[rs, :], xt_ref[rs, :]], axis=0)
        h = lax.dot_general(fw0, _bf(v), cdims,
                            preferred_element_type=jnp.float32)
        h = _leaky(h + fb0c)
        m = jnp.mean(h, axis=0, keepdims=True)
        hc = h - m
        vv = jnp.mean(hc * hc, axis=0, keepdims=True)
        hn = hc * (1.0 / jnp.sqrt(vv + 1e-5)) * fg + fb
        cf = lax.dot_general(fw1r, _bf(hn), cdims,
                             preferred_element_type=jnp.float32) + fb1
        out_ref[rs, :] = y_ref[rs, :] * jnp.maximum(cf, 0.0)


def _coef(y_r, gi_r, xs_r, gj_r, xt_r, fw0, fb0c, fg, fb, fw1r, fb1):
    rsp = lambda: pl.BlockSpec((CB, CW), lambda i: (i, 0))
    wspec = lambda r, c: pl.BlockSpec((r, c), lambda i: (0, 0))
    return pl.pallas_call(
        _coef_body,
        grid=(CROWS // CB,),
        in_specs=[rsp(), rsp(), rsp(), rsp(), rsp(),
                  wspec(64, 5), wspec(64, 1), wspec(64, 1), wspec(64, 1),
                  wspec(1, 64), wspec(1, 1)],
        out_specs=rsp(),
        out_shape=jax.ShapeDtypeStruct((CROWS, CW), jnp.float32),
        compiler_params=pltpu.CompilerParams(
            dimension_semantics=("parallel",)),
    )(y_r, gi_r, xs_r, gj_r, xt_r, fw0, fb0c, fg, fb, fw1r, fb1)


def kernel(emb_s, emb_t, at_s, at_t, x_s, x_t, edge_index, edge_weight,
           W0, b0, W1, b1, W2, b2, ln0_g, ln0_b, ln1_g, ln1_b,
           fW0, fb0, fln_g, fln_b, fW1, fb1):
    pad = PE - E_TOTAL
    zi = jnp.zeros((pad,), jnp.int32)
    zf = jnp.zeros((pad,), jnp.float32)
    srcf = jnp.concatenate([edge_index[0], zi])
    dstf = jnp.concatenate([edge_index[1], zi])
    ns_s, ns_t = _pre(emb_s, emb_t)
    gs, gt = _gather(emb_s, emb_t,
                     srcf.reshape(16, N_PH, PR, D),
                     dstf.reshape(16, N_PH, PR, D))
    src3 = srcf.reshape(16, RT, D)
    dst3 = dstf.reshape(16, RT, D)
    nse_s, x1e_s, nse_t, x1e_t = _scal(ns_s, ns_t, x_s[:, 1], x_t[:, 1],
                                       src3, dst3)
    ats_p = jnp.concatenate([at_s[1].reshape(-1),
                             jnp.zeros((pad * 8,), jnp.float32)])
    att_p = jnp.concatenate([at_t[1].reshape(-1),
                             jnp.zeros((pad * 8,), jnp.float32)])
    ew_p = jnp.concatenate([edge_weight[:, 0], zf]).reshape(PROWS, D)
    m8 = (jnp.arange(1024)[:, None] // 8
          == jnp.arange(D)[None, :]).astype(jnp.float32)
    v2 = lambda a: a.reshape(PROWS, D)
    col = lambda a: a.reshape(D, 1)
    bf16 = jnp.bfloat16
    y = _fnn(gs, gt, v2(nse_s), v2(nse_t), v2(x1e_s), v2(x1e_t),
             ats_p.reshape(PROWS, 1024), att_p.reshape(PROWS, 1024), ew_p, m8,
             W0[:, 0:D].astype(bf16), W0[:, D:2 * D].astype(bf16),
             W0[:, 2 * D:].astype(bf16), col(b0), col(ln0_g), col(ln0_b),
             W1.astype(bf16), col(b1), col(ln1_g), col(ln1_b),
             W2.astype(bf16), b2.reshape(1, 1))
    vc = lambda a: a.reshape(CROWS, CW)
    xs_c = vc(x1e_s)
    xt_c = vc(x1e_t)
    fargs = (fW0.astype(bf16), fb0.reshape(64, 1), fln_g.reshape(64, 1),
             fln_b.reshape(64, 1), fW1.astype(bf16), fb1.reshape(1, 1))
    for _ in range(2):
        gi3, gj3 = _seg(y.reshape(16, RT, D), src3, dst3)
        y = _coef(vc(y), vc(gi3), xs_c, vc(gj3), xt_c, *fargs)
    return y.reshape(PE)[:E_TOTAL]
